# Initial kernel scaffold; baseline (speedup 1.0000x reference)
import jax
import jax.numpy as jnp
from jax.experimental import pallas as pl


def kernel(x, vertices, adj, emb_table):
    raise NotImplementedError("write your pallas kernel here")


# speedup vs baseline: 1.7413x; 1.7413x over previous
import functools

import jax
import jax.numpy as jnp
from jax import lax
from jax.experimental import pallas as pl
from jax.experimental.pallas import tpu as pltpu
from jax.experimental.pallas import tpu_sc as plsc

B, N, D = 8, 2048, 128
NC, NS = 2, 16
NW = NC * NS
TOTAL = B * N
ROWS_PER_W = TOTAL // NW
CHUNK = 128
CHUNKS_PER_W = ROWS_PER_W // CHUNK


def _sc_gather(table, idx2d):
    mesh = plsc.VectorSubcoreMesh(core_axis_name="c", subcore_axis_name="s")

    @functools.partial(
        pl.kernel,
        mesh=mesh,
        out_type=jax.ShapeDtypeStruct((TOTAL, D), jnp.float32),
        scratch_types=[
            pltpu.VMEM((CHUNKS_PER_W, CHUNK), jnp.int32),
            pltpu.VMEM((ROWS_PER_W, D), jnp.float32),
            pltpu.SemaphoreType.DMA,
        ],
    )
    def k(table_hbm, idx_hbm, out_hbm, idx_v, rows_v, sem):
        wid = lax.axis_index("s") * NC + lax.axis_index("c")
        pltpu.sync_copy(idx_hbm.at[pl.ds(wid * CHUNKS_PER_W, CHUNKS_PER_W)], idx_v)
        copies = [
            pltpu.async_copy(
                table_hbm.at[idx_v.at[j]],
                rows_v.at[pl.ds(j * CHUNK, CHUNK)],
                sem,
            )
            for j in range(CHUNKS_PER_W)
        ]
        for c in copies:
            c.wait()
        pltpu.sync_copy(rows_v, out_hbm.at[pl.ds(wid * ROWS_PER_W, ROWS_PER_W)])

    return k(table, idx2d)


def _lsm_body(x_ref, e_ref, o_ref):
    xv = x_ref[0]
    ev = e_ref[0]
    for v, off in ((xv, 0), (ev, D)):
        m = jnp.max(v, axis=0, keepdims=True)
        lse = m + jnp.log(jnp.sum(jnp.exp(v - m), axis=0, keepdims=True))
        o_ref[0, :, off:off + D] = v - lse


def kernel(x, vertices, adj, emb_table):
    del adj
    idx2d = vertices.astype(jnp.int32).reshape(TOTAL // CHUNK, CHUNK)
    emb = _sc_gather(emb_table, idx2d).reshape(B, N, D)

    out = pl.pallas_call(
        _lsm_body,
        grid=(B,),
        in_specs=[
            pl.BlockSpec((1, N, D), lambda b: (b, 0, 0)),
            pl.BlockSpec((1, N, D), lambda b: (b, 0, 0)),
        ],
        out_specs=pl.BlockSpec((1, N, 2 * D), lambda b: (b, 0, 0)),
        out_shape=jax.ShapeDtypeStruct((B, N, 2 * D), jnp.float32),
    )(x, emb)
    return out
